# baseline (device time: 86063 ns/iter reference)
import functools

import jax
import jax.numpy as jnp
from jax import lax
from jax.experimental import pallas as pl
from jax.experimental.pallas import tpu as pltpu

N_DEV = 16
LOG2_N = 4
B, SQ, SKV = 2, 256, 256
HQ_TOT, DH = 64, 64
H_LOC = HQ_TOT // N_DEV
BLK = 64
D_MODEL = 512
D_HEADS = H_LOC * DH


def kernel(x, Wq, K_ext, V_ext, Wo):
    def body(x_ref, wq_ref, k_ref, v_ref, wo_ref, out_ref,
             recv_ref, send_sems, recv_sems):
        my = lax.axis_index("i")

        barrier = pltpu.get_barrier_semaphore()
        for k in range(LOG2_N):
            pl.semaphore_signal(
                barrier, inc=1,
                device_id=(my ^ (1 << k),),
                device_id_type=pl.DeviceIdType.MESH,
            )
        pl.semaphore_wait(barrier, LOG2_N)

        wq = wq_ref[:, pl.ds(my * D_HEADS, D_HEADS)].astype(jnp.bfloat16)
        wo = wo_ref[pl.ds(my * D_HEADS, D_HEADS), :].astype(jnp.bfloat16)

        row_blk = lax.broadcasted_iota(jnp.int32, (SQ, SKV), 0) // BLK
        col_blk = lax.broadcasted_iota(jnp.int32, (SQ, SKV), 1) // BLK
        mask = col_blk <= row_blk

        for b in range(B):
            xb = x_ref[b].astype(jnp.bfloat16)
            q = jnp.dot(xb, wq, preferred_element_type=jnp.float32)
            ctx_heads = []
            for h in range(H_LOC):
                qh = q[:, h * DH:(h + 1) * DH].astype(jnp.bfloat16)
                kh = k_ref[b, :, h, :].astype(jnp.bfloat16)
                vh = v_ref[b, :, h, :].astype(jnp.bfloat16)
                s = lax.dot_general(
                    qh, kh, (((1,), (1,)), ((), ())),
                    preferred_element_type=jnp.float32,
                ) * 0.125
                s = jnp.where(mask, s, -1e9)
                m = jnp.max(s, axis=-1, keepdims=True)
                w = jnp.exp(s - m)
                w = w / jnp.sum(w, axis=-1, keepdims=True)
                ctx_heads.append(jnp.dot(
                    w.astype(jnp.bfloat16), vh,
                    preferred_element_type=jnp.float32,
                ))
            ctx = jnp.concatenate(ctx_heads, axis=1).astype(jnp.bfloat16)
            out_ref[b] = jnp.dot(ctx, wo, preferred_element_type=jnp.float32)

        for k in range(LOG2_N):
            partner = my ^ (1 << k)
            rdma = pltpu.make_async_remote_copy(
                src_ref=out_ref,
                dst_ref=recv_ref.at[k],
                send_sem=send_sems.at[k],
                recv_sem=recv_sems.at[k],
                device_id=(partner,),
                device_id_type=pl.DeviceIdType.MESH,
            )
            rdma.start()
            rdma.wait()
            out_ref[...] = out_ref[...] + recv_ref[k]

        @functools.partial(pl.run_scoped, sem=pltpu.SemaphoreType.REGULAR)
        def _(sem):
            for k in range(LOG2_N):
                pl.semaphore_signal(
                    sem, inc=1,
                    device_id=(my ^ (1 << k),),
                    device_id_type=pl.DeviceIdType.MESH,
                )
            pl.semaphore_wait(sem, LOG2_N)

    return pl.pallas_call(
        body,
        out_shape=jax.ShapeDtypeStruct((B, SQ, D_MODEL), jnp.float32),
        in_specs=[pl.BlockSpec(memory_space=pltpu.VMEM)] * 5,
        out_specs=pl.BlockSpec(memory_space=pltpu.VMEM),
        scratch_shapes=[
            pltpu.VMEM((LOG2_N, B, SQ, D_MODEL), jnp.float32),
            pltpu.SemaphoreType.DMA((LOG2_N,)),
            pltpu.SemaphoreType.DMA((LOG2_N,)),
        ],
        compiler_params=pltpu.CompilerParams(collective_id=0),
    )(x, Wq, K_ext, V_ext, Wo)


# device time: 16725 ns/iter; 5.1458x vs baseline; 5.1458x over previous
import jax
import jax.numpy as jnp
from jax import lax
from jax.experimental import pallas as pl
from jax.experimental.pallas import tpu as pltpu

N_DEV = 16
B, SQ, SKV = 2, 256, 256
HQ_TOT, DH = 64, 64
H_LOC = HQ_TOT // N_DEV
BLK = 64
D_MODEL = 512
D_HEADS = H_LOC * DH
ROWS = B * SQ
SEG = ROWS // N_DEV


def kernel(x, Wq, K_ext, V_ext, Wo):
    def body(x_ref, wq_ref, k_ref, v_ref, wo_ref, out_ref,
             pbf_ref, stage_ref, gbuf_ref, allout_ref,
             send1, recv1, send2, recv2):
        my = lax.axis_index("i")

        wq = wq_ref[:, pl.ds(my * D_HEADS, D_HEADS)].astype(jnp.bfloat16)
        wo = wo_ref[pl.ds(my * D_HEADS, D_HEADS), :].astype(jnp.bfloat16)

        row_blk = lax.broadcasted_iota(jnp.int32, (SQ, SKV), 0) // BLK
        col_blk = lax.broadcasted_iota(jnp.int32, (SQ, SKV), 1) // BLK
        mask = col_blk <= row_blk

        for b in range(B):
            xb = x_ref[b].astype(jnp.bfloat16)
            q = jnp.dot(xb, wq, preferred_element_type=jnp.float32)
            ctx_heads = []
            for h in range(H_LOC):
                qh = q[:, h * DH:(h + 1) * DH].astype(jnp.bfloat16)
                kh = k_ref[b, :, h, :].astype(jnp.bfloat16)
                vh = v_ref[b, :, h, :].astype(jnp.bfloat16)
                s = lax.dot_general(
                    qh, kh, (((1,), (1,)), ((), ())),
                    preferred_element_type=jnp.float32,
                ) * 0.125
                s = jnp.where(mask, s, -1e9)
                m = jnp.max(s, axis=-1, keepdims=True)
                w = jnp.exp(s - m)
                w = w / jnp.sum(w, axis=-1, keepdims=True)
                ctx_heads.append(jnp.dot(
                    w.astype(jnp.bfloat16), vh,
                    preferred_element_type=jnp.float32,
                ))
            ctx = jnp.concatenate(ctx_heads, axis=1).astype(jnp.bfloat16)
            part = jnp.dot(ctx, wo, preferred_element_type=jnp.float32)
            pbf_ref[pl.ds(b * SQ, SQ), :] = part.astype(jnp.bfloat16)

        barrier = pltpu.get_barrier_semaphore()
        for d in range(1, N_DEV):
            pl.semaphore_signal(
                barrier, inc=1,
                device_id=((my + d) % N_DEV,),
                device_id_type=pl.DeviceIdType.MESH,
            )
        pl.semaphore_wait(barrier, N_DEV - 1)

        sends1 = []
        for d in range(1, N_DEV):
            p = (my + d) % N_DEV
            r = pltpu.make_async_remote_copy(
                src_ref=pbf_ref.at[pl.ds(p * SEG, SEG)],
                dst_ref=stage_ref.at[pl.ds(my * SEG, SEG)],
                send_sem=send1.at[p],
                recv_sem=recv1.at[my],
                device_id=(p,),
                device_id_type=pl.DeviceIdType.MESH,
            )
            r.start()
            sends1.append(r)
        stage_ref[pl.ds(my * SEG, SEG), :] = pbf_ref[pl.ds(my * SEG, SEG), :]
        for d in range(1, N_DEV):
            q_src = (my + d) % N_DEV
            pltpu.make_async_remote_copy(
                src_ref=pbf_ref.at[pl.ds(0, SEG)],
                dst_ref=stage_ref.at[pl.ds(q_src * SEG, SEG)],
                send_sem=send1.at[q_src],
                recv_sem=recv1.at[q_src],
                device_id=(q_src,),
                device_id_type=pl.DeviceIdType.MESH,
            ).wait_recv()
        for r in sends1:
            r.wait_send()

        s = stage_ref[...].astype(jnp.float32)
        seg_sum = s.reshape(N_DEV, SEG, D_MODEL).sum(axis=0)
        gbuf_ref[...] = seg_sum.astype(jnp.bfloat16)

        sends2 = []
        for d in range(1, N_DEV):
            p = (my + d) % N_DEV
            r = pltpu.make_async_remote_copy(
                src_ref=gbuf_ref,
                dst_ref=allout_ref.at[pl.ds(my * SEG, SEG)],
                send_sem=send2.at[p],
                recv_sem=recv2.at[my],
                device_id=(p,),
                device_id_type=pl.DeviceIdType.MESH,
            )
            r.start()
            sends2.append(r)
        allout_ref[pl.ds(my * SEG, SEG), :] = gbuf_ref[...]
        for d in range(1, N_DEV):
            q_src = (my + d) % N_DEV
            pltpu.make_async_remote_copy(
                src_ref=gbuf_ref,
                dst_ref=allout_ref.at[pl.ds(q_src * SEG, SEG)],
                send_sem=send2.at[q_src],
                recv_sem=recv2.at[q_src],
                device_id=(q_src,),
                device_id_type=pl.DeviceIdType.MESH,
            ).wait_recv()
        for r in sends2:
            r.wait_send()

        out_ref[...] = allout_ref[...].astype(jnp.float32).reshape(
            B, SQ, D_MODEL)

    return pl.pallas_call(
        body,
        out_shape=jax.ShapeDtypeStruct((B, SQ, D_MODEL), jnp.float32),
        in_specs=[pl.BlockSpec(memory_space=pltpu.VMEM)] * 5,
        out_specs=pl.BlockSpec(memory_space=pltpu.VMEM),
        scratch_shapes=[
            pltpu.VMEM((ROWS, D_MODEL), jnp.bfloat16),
            pltpu.VMEM((ROWS, D_MODEL), jnp.bfloat16),
            pltpu.VMEM((SEG, D_MODEL), jnp.bfloat16),
            pltpu.VMEM((ROWS, D_MODEL), jnp.bfloat16),
            pltpu.SemaphoreType.DMA((N_DEV,)),
            pltpu.SemaphoreType.DMA((N_DEV,)),
            pltpu.SemaphoreType.DMA((N_DEV,)),
            pltpu.SemaphoreType.DMA((N_DEV,)),
        ],
        compiler_params=pltpu.CompilerParams(collective_id=0),
    )(x, Wq, K_ext, V_ext, Wo)
